# baseline (speedup 1.0000x reference)
import functools

import jax
import jax.numpy as jnp
from jax import lax
from jax.experimental import pallas as pl
from jax.experimental.pallas import tpu as pltpu
from jax.experimental.pallas import tpu_sc as plsc

N_NODES = 10000
N_EDGES = 320000
D_FEAT = 128
N_GROUPS = 16

NP = 10240
EP = 327680
NC = 2
NS = 16
NW = NC * NS

CHUNK = 128
CPW = EP // (NW * CHUNK)
EPW = CPW * CHUNK
NBUF = 4
STRIPE = NP // NS

INTERPRET = False


@functools.lru_cache(maxsize=1)
def _sc_mesh():
    return plsc.VectorSubcoreMesh(
        core_axis_name="c", subcore_axis_name="s",
        num_cores=NC, num_subcores=NS)


def _worker_id():
    return lax.axis_index("s") * NC + lax.axis_index("c")


def _zero_fill(ref, width):
    zeros16 = jnp.zeros((16,), jnp.float32)

    def body(i, _):
        for j in range(width // 16):
            ref[i, pl.ds(j * 16, 16)] = zeros16
        return 0

    lax.fori_loop(0, ref.shape[0], body, 0)


def _zero_stripe(acc, zbuf, start):
    nz = zbuf.shape[0]
    for k in range(STRIPE // nz):
        pltpu.sync_copy(zbuf, acc.at[pl.ds(start + k * nz, nz)])


def _deg_body(dst_hbm, out_hbm, ones_v, didx, acc, d0, d1):
    c = lax.axis_index("c")
    s = lax.axis_index("s")
    w = _worker_id()
    dsem = [d0, d1]

    _zero_fill(ones_v, D_FEAT)
    zstart = pl.multiple_of(s * STRIPE, 8)
    for k in range(STRIPE // CHUNK):
        pltpu.sync_copy(ones_v, acc.at[pl.ds(zstart + k * CHUNK, CHUNK)])

    ones16 = jnp.ones((16,), jnp.float32)

    def ones_body(i, _):
        for j in range(D_FEAT // 16):
            ones_v[i, pl.ds(j * 16, 16)] = ones16
        return 0
    lax.fori_loop(0, CHUNK, ones_body, 0)

    pltpu.sync_copy(dst_hbm.at[pl.ds(w * CPW, CPW)], didx)
    plsc.subcore_barrier()

    def fire(j, b):
        pltpu.async_copy(ones_v, acc.at[didx.at[j]], dsem[b], add=True)

    def wait(j, b):
        pltpu.make_async_copy(ones_v, acc.at[didx.at[j]], dsem[b]).wait()

    fire(0, 0)
    fire(1, 1)

    def chunk_body(g, _):
        for k in range(2):
            j = g * 2 + k
            wait(j, k)
            fire(j + 2, k)
        return 0

    lax.fori_loop(0, CPW // 2 - 1, chunk_body, 0)
    wait(CPW - 2, 0)
    wait(CPW - 1, 1)

    plsc.subcore_barrier()
    dstart = pl.multiple_of(c * NP + s * STRIPE, 8)
    pltpu.sync_copy(acc.at[pl.ds(zstart, STRIPE)],
                    out_hbm.at[pl.ds(dstart, STRIPE)])


def _deg_call(dst2):
    return pl.kernel(
        _deg_body,
        out_type=jax.ShapeDtypeStruct((2 * NP, D_FEAT), jnp.float32),
        mesh=_sc_mesh(),
        scratch_types=[
            pltpu.VMEM((CHUNK, D_FEAT), jnp.float32),
            pltpu.VMEM((CPW, CHUNK), jnp.int32),
            pltpu.VMEM_SHARED((NP, D_FEAT), jnp.float32),
            pltpu.SemaphoreType.DMA,
            pltpu.SemaphoreType.DMA,
        ],
        interpret=INTERPRET,
    )(dst2)


ACHUNK = 128
ACPW = EP // (NW * ACHUNK)
ACPB = ACPW // 2


def _agg_body(yp_hbm, src_hbm, dst_hbm, out_hbm, sidx, didx, rows, acc,
              g0, g1):
    c = lax.axis_index("c")
    s = lax.axis_index("s")
    w = _worker_id()
    gsem = [g0, g1]

    _zero_fill(rows.at[0], D_FEAT)
    zstart = pl.multiple_of(s * STRIPE, 8)
    for k in range(STRIPE // ACHUNK):
        pltpu.sync_copy(rows.at[0], acc.at[pl.ds(zstart + k * ACHUNK, ACHUNK)])
    plsc.subcore_barrier()

    def fire_g(j, b):
        pltpu.async_copy(yp_hbm.at[sidx.at[j]], rows.at[b], gsem[b])

    def wait_g(j, b):
        pltpu.make_async_copy(
            yp_hbm.at[sidx.at[j]], rows.at[b], gsem[b]).wait()

    for blk in range(2):
        cbase = w * ACPW + blk * ACPB
        pltpu.sync_copy(src_hbm.at[pl.ds(cbase, ACPB)], sidx)
        pltpu.sync_copy(dst_hbm.at[pl.ds(cbase, ACPB)], didx)

        for k in range(2):
            fire_g(k, k)

        def pair_body(g, _):
            for k in range(2):
                j = g * 2 + k
                wait_g(j, k)
                pltpu.sync_copy(rows.at[k], acc.at[didx.at[j]], add=True)
                fire_g(j + 2, k)
            return 0

        lax.fori_loop(0, ACPB // 2 - 1, pair_body, 0)

        for k in range(2):
            j = ACPB - 2 + k
            wait_g(j, k)
            pltpu.sync_copy(rows.at[k], acc.at[didx.at[j]], add=True)

    plsc.subcore_barrier()
    dstart = pl.multiple_of(c * NP + s * STRIPE, 8)
    pltpu.sync_copy(acc.at[pl.ds(zstart, STRIPE)],
                    out_hbm.at[pl.ds(dstart, STRIPE)])


def _agg_call(yp, src, dst):
    return pl.kernel(
        _agg_body,
        out_type=jax.ShapeDtypeStruct((2 * NP, D_FEAT), jnp.float32),
        mesh=_sc_mesh(),
        scratch_types=[
            pltpu.VMEM((ACPB, ACHUNK), jnp.int32),
            pltpu.VMEM((ACPB, ACHUNK), jnp.int32),
            pltpu.VMEM((2, ACHUNK, D_FEAT), jnp.float32),
            pltpu.VMEM_SHARED((NP, D_FEAT), jnp.float32),
            pltpu.SemaphoreType.DMA,
            pltpu.SemaphoreType.DMA,
        ],
        interpret=INTERPRET,
    )(yp, src, dst)


_BLK = 1024
_GRID = NP // _BLK


def _mm_u_body(x_ref, w_ref, o_ref):
    o_ref[...] = jnp.dot(x_ref[...], w_ref[...],
                         preferred_element_type=jnp.float32)


def _mm_u(x, W1):
    return pl.pallas_call(
        _mm_u_body,
        grid=(_GRID,),
        in_specs=[
            pl.BlockSpec((_BLK, D_FEAT), lambda i: (i, 0)),
            pl.BlockSpec((D_FEAT, D_FEAT), lambda i: (0, 0)),
        ],
        out_specs=pl.BlockSpec((_BLK, D_FEAT), lambda i: (i, 0)),
        out_shape=jax.ShapeDtypeStruct((NP, D_FEAT), jnp.float32),
        interpret=INTERPRET,
    )(x, W1)


def _mm_scale_body(d0, d1, u_ref, o_ref, dv_ref):
    deg = d0[:, 0:1] + d1[:, 0:1] + 1.0
    dinv = 1.0 / jnp.sqrt(deg)
    o_ref[...] = u_ref[...] * dinv
    dv_ref[...] = jnp.broadcast_to(dinv, (_BLK, 16))


def _mm_first(x, W1, degp):
    u = _mm_u(x, W1)
    return pl.pallas_call(
        _mm_scale_body,
        grid=(_GRID,),
        in_specs=[
            pl.BlockSpec((_BLK, D_FEAT), lambda i: (i, 0)),
            pl.BlockSpec((_BLK, D_FEAT), lambda i: (_GRID + i, 0)),
            pl.BlockSpec((_BLK, D_FEAT), lambda i: (i, 0)),
        ],
        out_specs=[
            pl.BlockSpec((_BLK, D_FEAT), lambda i: (i, 0)),
            pl.BlockSpec((_BLK, 16), lambda i: (i, 0)),
        ],
        out_shape=[
            jax.ShapeDtypeStruct((NP, D_FEAT), jnp.float32),
            jax.ShapeDtypeStruct((NP, 16), jnp.float32),
        ],
        interpret=INTERPRET,
    )(degp, degp, u)


def _mm_mid_body(dv, t0, t1, yp, b_ref, w_ref, o_ref):
    dinv = dv[:, 0:1]
    h = jnp.maximum(dinv * (t0[...] + t1[...] + yp[...]) + b_ref[...], 0.0)
    o_ref[...] = dinv * jnp.dot(h, w_ref[...],
                                preferred_element_type=jnp.float32)


def _mm_mid(tt, yp, dinv16, W, b):
    return pl.pallas_call(
        _mm_mid_body,
        grid=(_GRID,),
        in_specs=[
            pl.BlockSpec((_BLK, 16), lambda i: (i, 0)),
            pl.BlockSpec((_BLK, D_FEAT), lambda i: (i, 0)),
            pl.BlockSpec((_BLK, D_FEAT), lambda i: (_GRID + i, 0)),
            pl.BlockSpec((_BLK, D_FEAT), lambda i: (i, 0)),
            pl.BlockSpec((1, D_FEAT), lambda i: (0, 0)),
            pl.BlockSpec((D_FEAT, D_FEAT), lambda i: (0, 0)),
        ],
        out_specs=pl.BlockSpec((_BLK, D_FEAT), lambda i: (i, 0)),
        out_shape=jax.ShapeDtypeStruct((NP, D_FEAT), jnp.float32),
        interpret=INTERPRET,
    )(dinv16, tt, tt, yp, b.reshape(1, D_FEAT), W)


def _final_body(dv, t0, t1, yp, b_ref, batch_ref, l1w, l1b, l2w, l2b,
                o_ref, sums, cnts):
    i = pl.program_id(0)

    @pl.when(i == 0)
    def _():
        sums[...] = jnp.zeros_like(sums)
        cnts[...] = jnp.zeros_like(cnts)

    dinv = dv[:, 0:1]
    h = jnp.maximum(dinv * (t0[...] + t1[...] + yp[...]) + b_ref[...], 0.0)

    brow = batch_ref[0]
    gids = lax.broadcasted_iota(jnp.int32, (N_GROUPS, _BLK), 0)
    onehot = (gids == brow).astype(jnp.float32)
    sums[...] += jnp.dot(onehot, h, preferred_element_type=jnp.float32)
    cnts[...] += jnp.sum(onehot, axis=1, keepdims=True)

    @pl.when(i == _GRID - 1)
    def _():
        pooled = sums[...] / jnp.maximum(cnts[...], 1.0)
        z = jnp.maximum(
            jnp.dot(pooled, l1w[...], preferred_element_type=jnp.float32)
            + l1b[...], 0.0)
        logits = jnp.dot(z, l2w[...], preferred_element_type=jnp.float32) \
            + l2b[...]
        m = jnp.max(logits, axis=1, keepdims=True)
        e = jnp.exp(logits - m)
        o_ref[...] = logits - m - jnp.log(jnp.sum(e, axis=1, keepdims=True))


def _final(tt, yp, dinv16, b, batch_r, L1W, L1b, L2W, L2b):
    return pl.pallas_call(
        _final_body,
        grid=(_GRID,),
        in_specs=[
            pl.BlockSpec((_BLK, 16), lambda i: (i, 0)),
            pl.BlockSpec((_BLK, D_FEAT), lambda i: (i, 0)),
            pl.BlockSpec((_BLK, D_FEAT), lambda i: (_GRID + i, 0)),
            pl.BlockSpec((_BLK, D_FEAT), lambda i: (i, 0)),
            pl.BlockSpec((1, D_FEAT), lambda i: (0, 0)),
            pl.BlockSpec((1, 1, _BLK), lambda i: (i, 0, 0)),
            pl.BlockSpec((D_FEAT, D_FEAT), lambda i: (0, 0)),
            pl.BlockSpec((1, D_FEAT), lambda i: (0, 0)),
            pl.BlockSpec((D_FEAT, 64), lambda i: (0, 0)),
            pl.BlockSpec((1, 64), lambda i: (0, 0)),
        ],
        out_specs=pl.BlockSpec((N_GROUPS, 64), lambda i: (0, 0)),
        out_shape=jax.ShapeDtypeStruct((N_GROUPS, 64), jnp.float32),
        scratch_shapes=[
            pltpu.VMEM((N_GROUPS, D_FEAT), jnp.float32),
            pltpu.VMEM((N_GROUPS, 1), jnp.float32),
        ],
        interpret=INTERPRET,
    )(dinv16, tt, tt, yp, b.reshape(1, D_FEAT), batch_r,
      L1W, L1b.reshape(1, D_FEAT), L2W, L2b.reshape(1, 64))


def kernel(x, edge_index, batch, W1, b1, Wc, bc, L1W, L1b, L2W, L2b):
    npad = NP - N_NODES
    epad = EP - N_EDGES
    pad_rows = (N_NODES + jnp.arange(epad, dtype=jnp.int32) % npad)
    src1 = jnp.concatenate([edge_index[0].astype(jnp.int32), pad_rows])
    dst1 = jnp.concatenate([edge_index[1].astype(jnp.int32), pad_rows])
    src = src1.reshape(EP // ACHUNK, ACHUNK)
    dst = dst1.reshape(EP // ACHUNK, ACHUNK)
    dst_deg = dst1.reshape(EP // CHUNK, CHUNK)
    x_p = jnp.concatenate([x, jnp.zeros((npad, D_FEAT), jnp.float32)])
    batch_r = jnp.concatenate(
        [batch.astype(jnp.int32),
         jnp.full((npad,), N_GROUPS, jnp.int32)]).reshape(_GRID, 1, _BLK)

    degp = _deg_call(dst_deg)
    yp, dinv16 = _mm_first(x_p, W1, degp)
    biases = [b1, bc[0], bc[1]]
    for i in range(3):
        tt = _agg_call(yp, src, dst)
        yp = _mm_mid(tt, yp, dinv16, Wc[i], biases[i])
    tt = _agg_call(yp, src, dst)
    return _final(tt, yp, dinv16, bc[2], batch_r, L1W, L1b, L2W, L2b)


# speedup vs baseline: 1.0025x; 1.0025x over previous
import functools

import jax
import jax.numpy as jnp
from jax import lax
from jax.experimental import pallas as pl
from jax.experimental.pallas import tpu as pltpu
from jax.experimental.pallas import tpu_sc as plsc

N_NODES = 10000
N_EDGES = 320000
D_FEAT = 128
N_GROUPS = 16

NP = 10240
EP = 327680
NC = 2
NS = 16
NW = NC * NS

CHUNK = 128
CPW = EP // (NW * CHUNK)
STRIPE = NP // NS

INTERPRET = False


@functools.lru_cache(maxsize=1)
def _sc_mesh():
    return plsc.VectorSubcoreMesh(
        core_axis_name="c", subcore_axis_name="s",
        num_cores=NC, num_subcores=NS)


def _worker_id():
    return lax.axis_index("s") * NC + lax.axis_index("c")


def _zero_fill(ref, width):
    zeros16 = jnp.zeros((16,), jnp.float32)

    def body(i, _):
        for j in range(width // 16):
            ref[i, pl.ds(j * 16, 16)] = zeros16
        return 0

    lax.fori_loop(0, ref.shape[0], body, 0)


def _deg_body(dst_hbm, out_hbm, ones_v, didx, acc, d0, d1):
    c = lax.axis_index("c")
    s = lax.axis_index("s")
    w = _worker_id()
    dsem = [d0, d1]

    _zero_fill(ones_v, D_FEAT)
    zstart = pl.multiple_of(s * STRIPE, 8)
    for k in range(STRIPE // CHUNK):
        pltpu.sync_copy(ones_v, acc.at[pl.ds(zstart + k * CHUNK, CHUNK)])

    ones16 = jnp.ones((16,), jnp.float32)

    def ones_body(i, _):
        for j in range(D_FEAT // 16):
            ones_v[i, pl.ds(j * 16, 16)] = ones16
        return 0
    lax.fori_loop(0, CHUNK, ones_body, 0)

    pltpu.sync_copy(dst_hbm.at[pl.ds(w * CPW, CPW)], didx)
    plsc.subcore_barrier()

    def fire(j, b):
        pltpu.async_copy(ones_v, acc.at[didx.at[j]], dsem[b], add=True)

    def wait(j, b):
        pltpu.make_async_copy(ones_v, acc.at[didx.at[j]], dsem[b]).wait()

    fire(0, 0)
    fire(1, 1)

    def chunk_body(g, _):
        for k in range(2):
            j = g * 2 + k
            wait(j, k)
            fire(j + 2, k)
        return 0

    lax.fori_loop(0, CPW // 2 - 1, chunk_body, 0)
    wait(CPW - 2, 0)
    wait(CPW - 1, 1)

    plsc.subcore_barrier()
    dstart = pl.multiple_of(c * NP + s * STRIPE, 8)
    pltpu.sync_copy(acc.at[pl.ds(zstart, STRIPE)],
                    out_hbm.at[pl.ds(dstart, STRIPE)])


def _deg_call(dst2):
    return pl.kernel(
        _deg_body,
        out_type=jax.ShapeDtypeStruct((2 * NP, D_FEAT), jnp.float32),
        mesh=_sc_mesh(),
        scratch_types=[
            pltpu.VMEM((CHUNK, D_FEAT), jnp.float32),
            pltpu.VMEM((CPW, CHUNK), jnp.int32),
            pltpu.VMEM_SHARED((NP, D_FEAT), jnp.float32),
            pltpu.SemaphoreType.DMA,
            pltpu.SemaphoreType.DMA,
        ],
        interpret=INTERPRET,
    )(dst2)


ACHUNK = 128
ACPW = EP // (NW * ACHUNK)
ACPB = ACPW // 2


def _agg_body(yp_hbm, src_hbm, dst_hbm, out_hbm, sidx, didx, rows, acc,
              g0, g1):
    c = lax.axis_index("c")
    s = lax.axis_index("s")
    w = _worker_id()
    gsem = [g0, g1]

    _zero_fill(rows.at[0], D_FEAT)
    zstart = pl.multiple_of(s * STRIPE, 8)
    for k in range(STRIPE // ACHUNK):
        pltpu.sync_copy(rows.at[0], acc.at[pl.ds(zstart + k * ACHUNK, ACHUNK)])
    plsc.subcore_barrier()

    def fire_g(j, b):
        pltpu.async_copy(yp_hbm.at[sidx.at[j]], rows.at[b], gsem[b])

    def wait_g(j, b):
        pltpu.make_async_copy(
            yp_hbm.at[sidx.at[j]], rows.at[b], gsem[b]).wait()

    for blk in range(2):
        cbase = w * ACPW + blk * ACPB
        pltpu.sync_copy(src_hbm.at[pl.ds(cbase, ACPB)], sidx)
        pltpu.sync_copy(dst_hbm.at[pl.ds(cbase, ACPB)], didx)

        for k in range(2):
            fire_g(k, k)

        def pair_body(g, _):
            for k in range(2):
                j = g * 2 + k
                wait_g(j, k)
                pltpu.sync_copy(rows.at[k], acc.at[didx.at[j]], add=True)
                fire_g(j + 2, k)
            return 0

        lax.fori_loop(0, ACPB // 2 - 1, pair_body, 0)

        for k in range(2):
            j = ACPB - 2 + k
            wait_g(j, k)
            pltpu.sync_copy(rows.at[k], acc.at[didx.at[j]], add=True)

    plsc.subcore_barrier()
    dstart = pl.multiple_of(c * NP + s * STRIPE, 8)
    pltpu.sync_copy(acc.at[pl.ds(zstart, STRIPE)],
                    out_hbm.at[pl.ds(dstart, STRIPE)])


def _agg_call(yp, src, dst):
    return pl.kernel(
        _agg_body,
        out_type=jax.ShapeDtypeStruct((2 * NP, D_FEAT), jnp.float32),
        mesh=_sc_mesh(),
        scratch_types=[
            pltpu.VMEM((ACPB, ACHUNK), jnp.int32),
            pltpu.VMEM((ACPB, ACHUNK), jnp.int32),
            pltpu.VMEM((2, ACHUNK, D_FEAT), jnp.float32),
            pltpu.VMEM_SHARED((NP, D_FEAT), jnp.float32),
            pltpu.SemaphoreType.DMA,
            pltpu.SemaphoreType.DMA,
        ],
        interpret=INTERPRET,
    )(yp, src, dst)


_BLK = 1024
_GRID = NP // _BLK


def _mm_first_body(d0, d1, x_ref, w_ref, o_ref, dv_ref):
    deg = d0[:, 0:1] + d1[:, 0:1] + 1.0
    dinv = 1.0 / jnp.sqrt(deg)
    y = jnp.dot(x_ref[...], w_ref[...], preferred_element_type=jnp.float32)
    o_ref[...] = y * dinv
    dv_ref[...] = jnp.broadcast_to(dinv, (_BLK, 16))


def _mm_first(x, W1, degp):
    return pl.pallas_call(
        _mm_first_body,
        grid=(_GRID,),
        in_specs=[
            pl.BlockSpec((_BLK, D_FEAT), lambda i: (i, 0)),
            pl.BlockSpec((_BLK, D_FEAT), lambda i: (_GRID + i, 0)),
            pl.BlockSpec((_BLK, D_FEAT), lambda i: (i, 0)),
            pl.BlockSpec((D_FEAT, D_FEAT), lambda i: (0, 0)),
        ],
        out_specs=[
            pl.BlockSpec((_BLK, D_FEAT), lambda i: (i, 0)),
            pl.BlockSpec((_BLK, 16), lambda i: (i, 0)),
        ],
        out_shape=[
            jax.ShapeDtypeStruct((NP, D_FEAT), jnp.float32),
            jax.ShapeDtypeStruct((NP, 16), jnp.float32),
        ],
        interpret=INTERPRET,
    )(degp, degp, x, W1)


def _mm_mid_body(dv, t0, t1, yp, b_ref, w_ref, o_ref):
    dinv = dv[:, 0:1]
    h = jnp.maximum(dinv * (t0[...] + t1[...] + yp[...]) + b_ref[...], 0.0)
    o_ref[...] = dinv * jnp.dot(h, w_ref[...],
                                preferred_element_type=jnp.float32)


def _mm_mid(tt, yp, dinv16, W, b):
    return pl.pallas_call(
        _mm_mid_body,
        grid=(_GRID,),
        in_specs=[
            pl.BlockSpec((_BLK, 16), lambda i: (i, 0)),
            pl.BlockSpec((_BLK, D_FEAT), lambda i: (i, 0)),
            pl.BlockSpec((_BLK, D_FEAT), lambda i: (_GRID + i, 0)),
            pl.BlockSpec((_BLK, D_FEAT), lambda i: (i, 0)),
            pl.BlockSpec((1, D_FEAT), lambda i: (0, 0)),
            pl.BlockSpec((D_FEAT, D_FEAT), lambda i: (0, 0)),
        ],
        out_specs=pl.BlockSpec((_BLK, D_FEAT), lambda i: (i, 0)),
        out_shape=jax.ShapeDtypeStruct((NP, D_FEAT), jnp.float32),
        interpret=INTERPRET,
    )(dinv16, tt, tt, yp, b.reshape(1, D_FEAT), W)


def _final_body(dv, t0, t1, yp, b_ref, batch_ref, l1w, l1b, l2w, l2b,
                o_ref, sums, cnts):
    i = pl.program_id(0)

    @pl.when(i == 0)
    def _():
        sums[...] = jnp.zeros_like(sums)
        cnts[...] = jnp.zeros_like(cnts)

    dinv = dv[:, 0:1]
    h = jnp.maximum(dinv * (t0[...] + t1[...] + yp[...]) + b_ref[...], 0.0)

    brow = batch_ref[0]
    gids = lax.broadcasted_iota(jnp.int32, (N_GROUPS, _BLK), 0)
    onehot = (gids == brow).astype(jnp.float32)
    sums[...] += jnp.dot(onehot, h, preferred_element_type=jnp.float32)
    cnts[...] += jnp.sum(onehot, axis=1, keepdims=True)

    @pl.when(i == _GRID - 1)
    def _():
        pooled = sums[...] / jnp.maximum(cnts[...], 1.0)
        z = jnp.maximum(
            jnp.dot(pooled, l1w[...], preferred_element_type=jnp.float32)
            + l1b[...], 0.0)
        logits = jnp.dot(z, l2w[...], preferred_element_type=jnp.float32) \
            + l2b[...]
        m = jnp.max(logits, axis=1, keepdims=True)
        e = jnp.exp(logits - m)
        o_ref[...] = logits - m - jnp.log(jnp.sum(e, axis=1, keepdims=True))


def _final(tt, yp, dinv16, b, batch_r, L1W, L1b, L2W, L2b):
    return pl.pallas_call(
        _final_body,
        grid=(_GRID,),
        in_specs=[
            pl.BlockSpec((_BLK, 16), lambda i: (i, 0)),
            pl.BlockSpec((_BLK, D_FEAT), lambda i: (i, 0)),
            pl.BlockSpec((_BLK, D_FEAT), lambda i: (_GRID + i, 0)),
            pl.BlockSpec((_BLK, D_FEAT), lambda i: (i, 0)),
            pl.BlockSpec((1, D_FEAT), lambda i: (0, 0)),
            pl.BlockSpec((1, 1, _BLK), lambda i: (i, 0, 0)),
            pl.BlockSpec((D_FEAT, D_FEAT), lambda i: (0, 0)),
            pl.BlockSpec((1, D_FEAT), lambda i: (0, 0)),
            pl.BlockSpec((D_FEAT, 64), lambda i: (0, 0)),
            pl.BlockSpec((1, 64), lambda i: (0, 0)),
        ],
        out_specs=pl.BlockSpec((N_GROUPS, 64), lambda i: (0, 0)),
        out_shape=jax.ShapeDtypeStruct((N_GROUPS, 64), jnp.float32),
        scratch_shapes=[
            pltpu.VMEM((N_GROUPS, D_FEAT), jnp.float32),
            pltpu.VMEM((N_GROUPS, 1), jnp.float32),
        ],
        interpret=INTERPRET,
    )(dinv16, tt, tt, yp, b.reshape(1, D_FEAT), batch_r,
      L1W, L1b.reshape(1, D_FEAT), L2W, L2b.reshape(1, 64))


def kernel(x, edge_index, batch, W1, b1, Wc, bc, L1W, L1b, L2W, L2b):
    npad = NP - N_NODES
    epad = EP - N_EDGES
    pad_rows = (N_NODES + jnp.arange(epad, dtype=jnp.int32) % npad)
    src1 = jnp.concatenate([edge_index[0].astype(jnp.int32), pad_rows])
    dst1 = jnp.concatenate([edge_index[1].astype(jnp.int32), pad_rows])
    src = src1.reshape(EP // ACHUNK, ACHUNK)
    dst = dst1.reshape(EP // ACHUNK, ACHUNK)
    dst_deg = dst1.reshape(EP // CHUNK, CHUNK)
    x_p = jnp.concatenate([x, jnp.zeros((npad, D_FEAT), jnp.float32)])
    batch_r = jnp.concatenate(
        [batch.astype(jnp.int32),
         jnp.full((npad,), N_GROUPS, jnp.int32)]).reshape(_GRID, 1, _BLK)

    degp = _deg_call(dst_deg)
    yp, dinv16 = _mm_first(x_p, W1, degp)
    biases = [b1, bc[0], bc[1]]
    for i in range(3):
        tt = _agg_call(yp, src, dst)
        yp = _mm_mid(tt, yp, dinv16, Wc[i], biases[i])
    tt = _agg_call(yp, src, dst)
    return _final(tt, yp, dinv16, bc[2], batch_r, L1W, L1b, L2W, L2b)
